# baseline (speedup 1.0000x reference)
import jax
import jax.numpy as jnp
from jax import lax
from jax.experimental import pallas as pl
from jax.experimental.pallas import tpu as pltpu
from jax.experimental.pallas import tpu_sc as plsc

N_USERS = 25000
N_ITEMS = 25000
N = N_USERS + N_ITEMS
D = 64
DH = D // 2
N_LAYERS = 3
E = 800000
B = 4096

NC = 2
NS = 16
L = 16

CHUNK = 128
OUTER = 49
IB = 8
CT = OUTER * IB
CH = CT * NS
E_PAD = CH * CHUNK
NP = 50176
ROWS_PER_TILE = NP // NS
ZR = 112
BT = B // NS
UH = BT // CHUNK


def _body(emb_cat, pack, wp, uix, iix,
          gamma_out, ping_a, ping_b,
          acc,
          idx_blk, w_blk, rows, rows2, rows3,
          u_idx, i_idx, gamma_v, gsem, ssem):
    c = lax.axis_index("c")
    t = lax.axis_index("s")
    f32 = jnp.float32
    bufs = (rows, rows2, rows3)

    pltpu.sync_copy(uix.at[c, pl.ds(UH * t, UH)], u_idx)
    pltpu.sync_copy(iix.at[c, pl.ds(UH * t, UH)], i_idx)

    base_row = t * ROWS_PER_TILE
    cbase = t * CT

    def scale(j, buf):
        def _m(g, _):
            w16 = w_blk[j, pl.ds(g * L, L)]
            for lane in range(L):
                r = g * L + lane
                wv = jnp.full((L,), w16[lane], dtype=f32)
                buf[r, pl.ds(0, L)] = buf[r, pl.ds(0, L)] * wv
                buf[r, pl.ds(L, L)] = buf[r, pl.ds(L, L)] * wv
            return 0
        lax.fori_loop(0, CHUNK // L, _m, 0, unroll=2)

    def add_rows(dst, src, nrows):
        def _a(r, _):
            dst[r, pl.ds(0, L)] = dst[r, pl.ds(0, L)] + src[r, pl.ds(0, L)]
            dst[r, pl.ds(L, L)] = dst[r, pl.ds(L, L)] + src[r, pl.ds(L, L)]
            return 0
        lax.fori_loop(0, nrows, _a, 0, unroll=2)

    tables = [emb_cat, ping_a, ping_b]
    for k in range(N_LAYERS):
        table = tables[k]

        def _zb(r, _):
            z = jnp.zeros((L,), f32)
            rows[r, pl.ds(0, L)] = z
            rows[r, pl.ds(L, L)] = z
            return 0
        lax.fori_loop(0, ZR, _zb, 0)

        zd = []
        for z in range(ROWS_PER_TILE // ZR):
            zd.append(pltpu.async_copy(
                rows.at[pl.ds(0, ZR)],
                acc.at[pl.ds(base_row + z * ZR, ZR)], ssem))
        for d in zd:
            d.wait()
        plsc.subcore_barrier()

        def _blk(ob, _):
            co = cbase + ob * IB
            pltpu.sync_copy(pack.at[c, pl.ds(co, IB)], idx_blk)
            pltpu.sync_copy(wp.at[pl.ds(co, IB)], w_blk)
            gd = {}
            sd = {}
            gd[0] = pltpu.async_copy(table.at[idx_blk.at[0, 0]], bufs[0], gsem)
            gd[1] = pltpu.async_copy(table.at[idx_blk.at[1, 0]], bufs[1], gsem)
            for j in range(IB):
                gd[j].wait()
                if j + 2 < IB:
                    if j >= 1:
                        sd[j - 1].wait()
                    gd[j + 2] = pltpu.async_copy(
                        table.at[idx_blk.at[j + 2, 0]], bufs[(j + 2) % 3], gsem)
                scale(j, bufs[j % 3])
                sd[j] = pltpu.async_copy(
                    bufs[j % 3], acc.at[idx_blk.at[j, 1]], ssem, add=True)
            sd[IB - 3].wait()
            sd[IB - 2].wait()
            sd[IB - 1].wait()
            return 0
        lax.fori_loop(0, OUTER, _blk, 0)
        plsc.subcore_barrier()

        if k < N_LAYERS - 1:
            out_tab = ping_a if k == 0 else ping_b
            pltpu.sync_copy(
                acc.at[pl.ds(base_row, ROWS_PER_TILE)],
                out_tab.at[pl.ds(c * NP + base_row, ROWS_PER_TILE)])
            plsc.subcore_barrier()

    def _s(z, _):
        row0 = base_row + z * ZR
        hrow = c * NP + row0
        d0 = pltpu.async_copy(emb_cat.at[pl.ds(hrow, ZR)],
                              rows.at[pl.ds(0, ZR)], gsem)
        d1 = pltpu.async_copy(ping_a.at[pl.ds(hrow, ZR)],
                              rows2.at[pl.ds(0, ZR)], gsem)
        d2 = pltpu.async_copy(ping_b.at[pl.ds(hrow, ZR)],
                              rows3.at[pl.ds(0, ZR)], gsem)
        d0.wait()
        d1.wait()
        d2.wait()
        add_rows(rows, rows2, ZR)
        pltpu.sync_copy(acc.at[pl.ds(row0, ZR)], rows2.at[pl.ds(0, ZR)])
        add_rows(rows, rows3, ZR)
        add_rows(rows, rows2, ZR)
        pltpu.sync_copy(rows.at[pl.ds(0, ZR)], ping_a.at[pl.ds(hrow, ZR)])
        return 0
    lax.fori_loop(0, ROWS_PER_TILE // ZR, _s, 0)
    plsc.subcore_barrier()

    lane_iota = lax.iota(jnp.int32, L)
    perms = [lane_iota ^ kk for kk in (8, 4, 2, 1)]

    def hsum(v):
        for p in perms:
            v = v + jnp.take(v, p)
        return v

    for h in range(UH):
        du = pltpu.async_copy(ping_a.at[u_idx.at[h]], rows, gsem)
        di = pltpu.async_copy(ping_a.at[i_idx.at[h]], rows2, gsem)
        du.wait()
        di.wait()

        def _g(g, _):
            acc16 = jnp.zeros((L,), f32)
            for lane in range(L):
                b2 = g * L + lane
                s = (rows[b2, pl.ds(0, L)] * rows2[b2, pl.ds(0, L)]
                     + rows[b2, pl.ds(L, L)] * rows2[b2, pl.ds(L, L)])
                sv = hsum(s) * f32(1.0 / 16.0)
                acc16 = jnp.where(lane_iota == lane, sv, acc16)
            gamma_v[pl.ds(h * CHUNK + g * L, L)] = acc16
            return 0
        lax.fori_loop(0, CHUNK // L, _g, 0)
    pltpu.sync_copy(gamma_v, gamma_out.at[c, pl.ds(t * BT, BT)])


@jax.jit
def _sc_call(emb_cat, pack, wp, uix, iix):
    mesh = plsc.VectorSubcoreMesh(core_axis_name="c", subcore_axis_name="s")
    f32 = jnp.float32
    call = pl.kernel(
        _body,
        out_type=[
            jax.ShapeDtypeStruct((NC, B), f32),
            jax.ShapeDtypeStruct((NC * NP, DH), f32),
            jax.ShapeDtypeStruct((NC * NP, DH), f32),
        ],
        mesh=mesh,
        compiler_params=pltpu.CompilerParams(use_tc_tiling_on_sc=False),
        scratch_types=[
            pltpu.VMEM_SHARED((NP, DH), f32),
            pltpu.VMEM((IB, 2, CHUNK), jnp.int32),
            pltpu.VMEM((IB, CHUNK), f32),
            pltpu.VMEM((CHUNK, DH), f32),
            pltpu.VMEM((CHUNK, DH), f32),
            pltpu.VMEM((CHUNK, DH), f32),
            pltpu.VMEM((UH, CHUNK), jnp.int32),
            pltpu.VMEM((UH, CHUNK), jnp.int32),
            pltpu.VMEM((BT,), f32),
            pltpu.SemaphoreType.DMA,
            pltpu.SemaphoreType.DMA,
        ],
    )
    return call(emb_cat, pack, wp, uix, iix)


def kernel(user_emb, item_emb, edge_weight, users, items, edge_index):
    all_emb = jnp.concatenate([user_emb, item_emb], axis=0)
    zrows = jnp.zeros((NP - N, DH), jnp.float32)
    emb_cat = jnp.concatenate(
        [all_emb[:, :DH], zrows, all_emb[:, DH:], zrows], axis=0)

    src = edge_index[0]
    dst = edge_index[1]
    pad = E_PAD - E
    srcp = jnp.pad(src, (0, pad))
    dstp = jnp.pad(dst, (0, pad)).reshape(CH, CHUNK)
    wp = jnp.pad(edge_weight, (0, pad)).reshape(CH, CHUNK)
    srcp = jnp.stack([srcp, srcp + NP]).reshape(NC, CH, CHUNK)
    pack = jnp.stack(
        [srcp, jnp.broadcast_to(dstp, (NC, CH, CHUNK))], axis=2)

    uix = jnp.stack([users, users + NP]).reshape(NC, B // CHUNK, CHUNK)
    it = items + N_USERS
    iix = jnp.stack([it, it + NP]).reshape(NC, B // CHUNK, CHUNK)

    gamma_parts, _, _ = _sc_call(emb_cat, pack, wp, uix, iix)
    return gamma_parts[0] + gamma_parts[1]


# speedup vs baseline: 1.8168x; 1.8168x over previous
import jax
import jax.numpy as jnp
from jax import lax
from jax.experimental import pallas as pl
from jax.experimental.pallas import tpu as pltpu
from jax.experimental.pallas import tpu_sc as plsc

N_USERS = 25000
N_ITEMS = 25000
N = N_USERS + N_ITEMS
D = 64
DH = D // 2
N_LAYERS = 3
E = 800000
B = 4096

NC = 2
NS = 16
L = 16

CHUNK = 128
OUTER = 49
IB = 8
CT = OUTER * IB
CH = CT * NS
E_PAD = CH * CHUNK
NP = 50176
ROWS_PER_TILE = NP // NS
ZR = 112
BT = B // NS
UH = BT // CHUNK


def _body(emb_cat, pack, wp, uix, iix,
          gamma_out, ping_a, ping_b,
          acc,
          idx_blk, w_blk, rows, rows2, rows3,
          u_idx, i_idx, gamma_v, gsem, ssem):
    c = lax.axis_index("c")
    t = lax.axis_index("s")
    f32 = jnp.float32
    bufs = (rows, rows2, rows3)

    pltpu.sync_copy(uix.at[c, pl.ds(UH * t, UH)], u_idx)
    pltpu.sync_copy(iix.at[c, pl.ds(UH * t, UH)], i_idx)

    base_row = t * ROWS_PER_TILE
    cbase = t * CT

    def scale(j, buf):
        def _m(g, _):
            w16 = w_blk[j, pl.ds(g * L, L)]
            for lane in range(L):
                r = g * L + lane
                wv = jnp.full((L,), w16[lane], dtype=f32)
                buf[r, pl.ds(0, L)] = buf[r, pl.ds(0, L)] * wv
                buf[r, pl.ds(L, L)] = buf[r, pl.ds(L, L)] * wv
            return 0
        lax.fori_loop(0, CHUNK // L, _m, 0)

    def add_rows(dst, src, nrows):
        def _a(r, _):
            dst[r, pl.ds(0, L)] = dst[r, pl.ds(0, L)] + src[r, pl.ds(0, L)]
            dst[r, pl.ds(L, L)] = dst[r, pl.ds(L, L)] + src[r, pl.ds(L, L)]
            return 0
        lax.fori_loop(0, nrows, _a, 0)

    tables = [emb_cat, ping_a, ping_b]
    for k in range(N_LAYERS):
        table = tables[k]

        def _zb(r, _):
            z = jnp.zeros((L,), f32)
            rows[r, pl.ds(0, L)] = z
            rows[r, pl.ds(L, L)] = z
            return 0
        lax.fori_loop(0, ZR, _zb, 0)

        zd = []
        for z in range(ROWS_PER_TILE // ZR):
            zd.append(pltpu.async_copy(
                rows.at[pl.ds(0, ZR)],
                acc.at[pl.ds(base_row + z * ZR, ZR)], ssem))
        for d in zd:
            d.wait()
        plsc.subcore_barrier()

        def _blk(ob, _):
            co = cbase + ob * IB
            pltpu.sync_copy(pack.at[c, pl.ds(co, IB)], idx_blk)
            pltpu.sync_copy(wp.at[pl.ds(co, IB)], w_blk)
            gd = {}
            sd = {}
            gd[0] = pltpu.async_copy(table.at[idx_blk.at[0, 0]], bufs[0], gsem)
            gd[1] = pltpu.async_copy(table.at[idx_blk.at[1, 0]], bufs[1], gsem)
            for j in range(IB):
                gd[j].wait()
                if j + 2 < IB:
                    if j >= 1:
                        sd[j - 1].wait()
                    gd[j + 2] = pltpu.async_copy(
                        table.at[idx_blk.at[j + 2, 0]], bufs[(j + 2) % 3], gsem)
                scale(j, bufs[j % 3])
                sd[j] = pltpu.async_copy(
                    bufs[j % 3], acc.at[idx_blk.at[j, 1]], ssem, add=True)
            sd[IB - 3].wait()
            sd[IB - 2].wait()
            sd[IB - 1].wait()
            return 0
        lax.fori_loop(0, OUTER, _blk, 0)
        plsc.subcore_barrier()

        if k < N_LAYERS - 1:
            out_tab = ping_a if k == 0 else ping_b
            pltpu.sync_copy(
                acc.at[pl.ds(base_row, ROWS_PER_TILE)],
                out_tab.at[pl.ds(c * NP + base_row, ROWS_PER_TILE)])
            plsc.subcore_barrier()

    def _s(z, _):
        row0 = base_row + z * ZR
        hrow = c * NP + row0
        d0 = pltpu.async_copy(emb_cat.at[pl.ds(hrow, ZR)],
                              rows.at[pl.ds(0, ZR)], gsem)
        d1 = pltpu.async_copy(ping_a.at[pl.ds(hrow, ZR)],
                              rows2.at[pl.ds(0, ZR)], gsem)
        d2 = pltpu.async_copy(ping_b.at[pl.ds(hrow, ZR)],
                              rows3.at[pl.ds(0, ZR)], gsem)
        d0.wait()
        d1.wait()
        d2.wait()
        add_rows(rows, rows2, ZR)
        pltpu.sync_copy(acc.at[pl.ds(row0, ZR)], rows2.at[pl.ds(0, ZR)])
        add_rows(rows, rows3, ZR)
        add_rows(rows, rows2, ZR)
        pltpu.sync_copy(rows.at[pl.ds(0, ZR)], ping_a.at[pl.ds(hrow, ZR)])
        return 0
    lax.fori_loop(0, ROWS_PER_TILE // ZR, _s, 0)
    plsc.subcore_barrier()

    lane_iota = lax.iota(jnp.int32, L)
    perms = [lane_iota ^ kk for kk in (8, 4, 2, 1)]

    def hsum(v):
        for p in perms:
            v = v + jnp.take(v, p)
        return v

    for h in range(UH):
        du = pltpu.async_copy(ping_a.at[u_idx.at[h]], rows, gsem)
        di = pltpu.async_copy(ping_a.at[i_idx.at[h]], rows2, gsem)
        du.wait()
        di.wait()

        def _g(g, _):
            acc16 = jnp.zeros((L,), f32)
            for lane in range(L):
                b2 = g * L + lane
                s = (rows[b2, pl.ds(0, L)] * rows2[b2, pl.ds(0, L)]
                     + rows[b2, pl.ds(L, L)] * rows2[b2, pl.ds(L, L)])
                sv = hsum(s) * f32(1.0 / 16.0)
                acc16 = jnp.where(lane_iota == lane, sv, acc16)
            gamma_v[pl.ds(h * CHUNK + g * L, L)] = acc16
            return 0
        lax.fori_loop(0, CHUNK // L, _g, 0)
    pltpu.sync_copy(gamma_v, gamma_out.at[c, pl.ds(t * BT, BT)])


@jax.jit
def _sc_call(emb_cat, pack, wp, uix, iix):
    mesh = plsc.VectorSubcoreMesh(core_axis_name="c", subcore_axis_name="s")
    f32 = jnp.float32
    call = pl.kernel(
        _body,
        out_type=[
            jax.ShapeDtypeStruct((NC, B), f32),
            jax.ShapeDtypeStruct((NC * NP, DH), f32),
            jax.ShapeDtypeStruct((NC * NP, DH), f32),
        ],
        mesh=mesh,
        compiler_params=pltpu.CompilerParams(use_tc_tiling_on_sc=False),
        scratch_types=[
            pltpu.VMEM_SHARED((NP, DH), f32),
            pltpu.VMEM((IB, 2, CHUNK), jnp.int32),
            pltpu.VMEM((IB, CHUNK), f32),
            pltpu.VMEM((CHUNK, DH), f32),
            pltpu.VMEM((CHUNK, DH), f32),
            pltpu.VMEM((CHUNK, DH), f32),
            pltpu.VMEM((UH, CHUNK), jnp.int32),
            pltpu.VMEM((UH, CHUNK), jnp.int32),
            pltpu.VMEM((BT,), f32),
            pltpu.SemaphoreType.DMA,
            pltpu.SemaphoreType.DMA,
        ],
    )
    return call(emb_cat, pack, wp, uix, iix)


def kernel(user_emb, item_emb, edge_weight, users, items, edge_index):
    all_emb = jnp.concatenate([user_emb, item_emb], axis=0)
    zrows = jnp.zeros((NP - N, DH), jnp.float32)
    emb_cat = jnp.concatenate(
        [all_emb[:, :DH], zrows, all_emb[:, DH:], zrows], axis=0)

    src = edge_index[0]
    dst = edge_index[1]
    pad = E_PAD - E
    srcp = jnp.pad(src, (0, pad))
    dstp = jnp.pad(dst, (0, pad)).reshape(CH, CHUNK)
    wp = jnp.pad(edge_weight, (0, pad)).reshape(CH, CHUNK)
    srcp = jnp.stack([srcp, srcp + NP]).reshape(NC, CH, CHUNK)
    pack = jnp.stack(
        [srcp, jnp.broadcast_to(dstp, (NC, CH, CHUNK))], axis=2)

    uix = jnp.stack([users, users + NP]).reshape(NC, B // CHUNK, CHUNK)
    it = items + N_USERS
    iix = jnp.stack([it, it + NP]).reshape(NC, B // CHUNK, CHUNK)

    gamma_parts, _, _ = _sc_call(emb_cat, pack, wp, uix, iix)
    return gamma_parts[0] + gamma_parts[1]


# speedup vs baseline: 2.1685x; 1.1936x over previous
import jax
import jax.numpy as jnp
from jax import lax
from jax.experimental import pallas as pl
from jax.experimental.pallas import tpu as pltpu
from jax.experimental.pallas import tpu_sc as plsc

N_USERS = 25000
N_ITEMS = 25000
N = N_USERS + N_ITEMS
D = 64
DH = D // 2
N_LAYERS = 3
E = 800000
B = 4096

NC = 2
NS = 16
L = 16

CHUNK = 128
OUTER = 49
IB = 8
CT = OUTER * IB
CH = CT * NS
E_PAD = CH * CHUNK
NP = 50176
ROWS_PER_TILE = NP // NS
ZR = 112
BT = B // NS
UH = BT // CHUNK


def _body(emb_cat, pack, wp, uix, iix,
          gamma_out, ping_a, ping_b,
          acc,
          idx_blk, w_blk, rows, rows2, rows3, rows4,
          u_idx, i_idx, gamma_v, gsem, ssem, isem):
    c = lax.axis_index("c")
    t = lax.axis_index("s")
    f32 = jnp.float32
    bufs = (rows, rows2, rows3, rows4)

    pltpu.sync_copy(uix.at[c, pl.ds(UH * t, UH)], u_idx)
    pltpu.sync_copy(iix.at[c, pl.ds(UH * t, UH)], i_idx)

    base_row = t * ROWS_PER_TILE
    cbase = t * CT

    def scale(p, j, buf):
        def _m(g, _):
            w16 = w_blk[p, j, pl.ds(g * L, L)]
            for lane in range(L):
                r = g * L + lane
                wv = jnp.full((L,), w16[lane], dtype=f32)
                buf[r, pl.ds(0, L)] = buf[r, pl.ds(0, L)] * wv
                buf[r, pl.ds(L, L)] = buf[r, pl.ds(L, L)] * wv
            return 0
        lax.fori_loop(0, CHUNK // L, _m, 0)

    def add_rows(dst, src, nrows):
        def _a(r, _):
            dst[r, pl.ds(0, L)] = dst[r, pl.ds(0, L)] + src[r, pl.ds(0, L)]
            dst[r, pl.ds(L, L)] = dst[r, pl.ds(L, L)] + src[r, pl.ds(L, L)]
            return 0
        lax.fori_loop(0, nrows, _a, 0)

    tables = [emb_cat, ping_a, ping_b]
    for k in range(N_LAYERS):
        table = tables[k]

        def _zb(r, _):
            z = jnp.zeros((L,), f32)
            rows[r, pl.ds(0, L)] = z
            rows[r, pl.ds(L, L)] = z
            return 0
        lax.fori_loop(0, ZR, _zb, 0)

        zd = []
        for z in range(ROWS_PER_TILE // ZR):
            zd.append(pltpu.async_copy(
                rows.at[pl.ds(0, ZR)],
                acc.at[pl.ds(base_row + z * ZR, ZR)], ssem))
        for d in zd:
            d.wait()
        plsc.subcore_barrier()

        pltpu.sync_copy(pack.at[c, pl.ds(cbase, IB)], idx_blk.at[0])
        pltpu.sync_copy(wp.at[pl.ds(cbase, IB)], w_blk.at[0])

        def _blk(ob, _):
            p = ob % 2
            q = (ob + 1) % 2
            co = cbase + ob * IB

            @pl.when(ob + 1 < OUTER)
            def _pf():
                pltpu.async_copy(pack.at[c, pl.ds(co + IB, IB)],
                                 idx_blk.at[q], isem)
                pltpu.async_copy(wp.at[pl.ds(co + IB, IB)], w_blk.at[q], isem)

            gd = {}
            sd = {}
            gd[0] = pltpu.async_copy(table.at[idx_blk.at[p, 0, 0]], bufs[0], gsem)
            gd[1] = pltpu.async_copy(table.at[idx_blk.at[p, 1, 0]], bufs[1], gsem)
            for j in range(IB):
                gd[j].wait()
                if j + 2 < IB:
                    if j >= 2:
                        sd[j - 2].wait()
                    gd[j + 2] = pltpu.async_copy(
                        table.at[idx_blk.at[p, j + 2, 0]], bufs[(j + 2) % 4], gsem)
                scale(p, j, bufs[j % 4])
                sd[j] = pltpu.async_copy(
                    bufs[j % 4], acc.at[idx_blk.at[p, j, 1]], ssem, add=True)
            sd[IB - 4].wait()
            sd[IB - 3].wait()
            sd[IB - 2].wait()
            sd[IB - 1].wait()

            @pl.when(ob + 1 < OUTER)
            def _pfw():
                pltpu.make_async_copy(pack.at[c, pl.ds(co + IB, IB)],
                                      idx_blk.at[q], isem).wait()
                pltpu.make_async_copy(wp.at[pl.ds(co + IB, IB)],
                                      w_blk.at[q], isem).wait()
            return 0
        lax.fori_loop(0, OUTER, _blk, 0)
        plsc.subcore_barrier()

        if k < N_LAYERS - 1:
            out_tab = ping_a if k == 0 else ping_b
            pltpu.sync_copy(
                acc.at[pl.ds(base_row, ROWS_PER_TILE)],
                out_tab.at[pl.ds(c * NP + base_row, ROWS_PER_TILE)])
            plsc.subcore_barrier()

    def _s(z, _):
        row0 = base_row + z * ZR
        hrow = c * NP + row0
        d0 = pltpu.async_copy(emb_cat.at[pl.ds(hrow, ZR)],
                              rows.at[pl.ds(0, ZR)], gsem)
        d1 = pltpu.async_copy(ping_a.at[pl.ds(hrow, ZR)],
                              rows2.at[pl.ds(0, ZR)], gsem)
        d2 = pltpu.async_copy(ping_b.at[pl.ds(hrow, ZR)],
                              rows3.at[pl.ds(0, ZR)], gsem)
        d0.wait()
        d1.wait()
        d2.wait()
        add_rows(rows, rows2, ZR)
        pltpu.sync_copy(acc.at[pl.ds(row0, ZR)], rows2.at[pl.ds(0, ZR)])
        add_rows(rows, rows3, ZR)
        add_rows(rows, rows2, ZR)
        pltpu.sync_copy(rows.at[pl.ds(0, ZR)], ping_a.at[pl.ds(hrow, ZR)])
        return 0
    lax.fori_loop(0, ROWS_PER_TILE // ZR, _s, 0)
    plsc.subcore_barrier()

    lane_iota = lax.iota(jnp.int32, L)
    perms = [lane_iota ^ kk for kk in (8, 4, 2, 1)]

    def hsum(v):
        for p in perms:
            v = v + jnp.take(v, p)
        return v

    for h in range(UH):
        du = pltpu.async_copy(ping_a.at[u_idx.at[h]], rows, gsem)
        di = pltpu.async_copy(ping_a.at[i_idx.at[h]], rows2, gsem)
        du.wait()
        di.wait()

        def _g(g, _):
            acc16 = jnp.zeros((L,), f32)
            for lane in range(L):
                b2 = g * L + lane
                s = (rows[b2, pl.ds(0, L)] * rows2[b2, pl.ds(0, L)]
                     + rows[b2, pl.ds(L, L)] * rows2[b2, pl.ds(L, L)])
                sv = hsum(s) * f32(1.0 / 16.0)
                acc16 = jnp.where(lane_iota == lane, sv, acc16)
            gamma_v[pl.ds(h * CHUNK + g * L, L)] = acc16
            return 0
        lax.fori_loop(0, CHUNK // L, _g, 0)
    pltpu.sync_copy(gamma_v, gamma_out.at[c, pl.ds(t * BT, BT)])


@jax.jit
def _sc_call(emb_cat, pack, wp, uix, iix):
    mesh = plsc.VectorSubcoreMesh(core_axis_name="c", subcore_axis_name="s")
    f32 = jnp.float32
    call = pl.kernel(
        _body,
        out_type=[
            jax.ShapeDtypeStruct((NC, B), f32),
            jax.ShapeDtypeStruct((NC * NP, DH), f32),
            jax.ShapeDtypeStruct((NC * NP, DH), f32),
        ],
        mesh=mesh,
        compiler_params=pltpu.CompilerParams(use_tc_tiling_on_sc=False),
        scratch_types=[
            pltpu.VMEM_SHARED((NP, DH), f32),
            pltpu.VMEM((2, IB, 2, CHUNK), jnp.int32),
            pltpu.VMEM((2, IB, CHUNK), f32),
            pltpu.VMEM((CHUNK, DH), f32),
            pltpu.VMEM((CHUNK, DH), f32),
            pltpu.VMEM((CHUNK, DH), f32),
            pltpu.VMEM((CHUNK, DH), f32),
            pltpu.VMEM((UH, CHUNK), jnp.int32),
            pltpu.VMEM((UH, CHUNK), jnp.int32),
            pltpu.VMEM((BT,), f32),
            pltpu.SemaphoreType.DMA,
            pltpu.SemaphoreType.DMA,
            pltpu.SemaphoreType.DMA,
        ],
    )
    return call(emb_cat, pack, wp, uix, iix)


def kernel(user_emb, item_emb, edge_weight, users, items, edge_index):
    all_emb = jnp.concatenate([user_emb, item_emb], axis=0)
    zrows = jnp.zeros((NP - N, DH), jnp.float32)
    emb_cat = jnp.concatenate(
        [all_emb[:, :DH], zrows, all_emb[:, DH:], zrows], axis=0)

    src = edge_index[0]
    dst = edge_index[1]
    pad = E_PAD - E
    srcp = jnp.pad(src, (0, pad))
    dstp = jnp.pad(dst, (0, pad)).reshape(CH, CHUNK)
    wp = jnp.pad(edge_weight, (0, pad)).reshape(CH, CHUNK)
    srcp = jnp.stack([srcp, srcp + NP]).reshape(NC, CH, CHUNK)
    pack = jnp.stack(
        [srcp, jnp.broadcast_to(dstp, (NC, CH, CHUNK))], axis=2)

    uix = jnp.stack([users, users + NP]).reshape(NC, B // CHUNK, CHUNK)
    it = items + N_USERS
    iix = jnp.stack([it, it + NP]).reshape(NC, B // CHUNK, CHUNK)

    gamma_parts, _, _ = _sc_call(emb_cat, pack, wp, uix, iix)
    return gamma_parts[0] + gamma_parts[1]


# speedup vs baseline: 2.3361x; 1.0773x over previous
import jax
import jax.numpy as jnp
from jax import lax
from jax.experimental import pallas as pl
from jax.experimental.pallas import tpu as pltpu
from jax.experimental.pallas import tpu_sc as plsc

N_USERS = 25000
N_ITEMS = 25000
N = N_USERS + N_ITEMS
D = 64
DH = D // 2
N_LAYERS = 3
E = 800000
B = 4096

NC = 2
NS = 16
L = 16

CHUNK = 128
OUTER = 49
IB = 8
CT = OUTER * IB
CH = CT * NS
E_PAD = CH * CHUNK
NP = 50176
ROWS_PER_TILE = NP // NS
ZR = 112
BT = B // NS
UH = BT // CHUNK


def _body(emb_cat, pack, wp, uix, iix,
          gamma_out, ping_a, ping_b, ping_c,
          acc,
          idx_blk, w_blk, rows, rows2, rows3, rows4,
          u_idx, i_idx, gamma_v, gsem, ssem, isem):
    c = lax.axis_index("c")
    t = lax.axis_index("s")
    f32 = jnp.float32
    bufs = (rows, rows2, rows3, rows4)

    pltpu.sync_copy(uix.at[c, pl.ds(UH * t, UH)], u_idx)
    pltpu.sync_copy(iix.at[c, pl.ds(UH * t, UH)], i_idx)

    base_row = t * ROWS_PER_TILE
    cbase = t * CT

    def scale(p, j, buf):
        def _m(g, _):
            w16 = w_blk[p, j, pl.ds(g * L, L)]
            for lane in range(L):
                r = g * L + lane
                wv = jnp.full((L,), w16[lane], dtype=f32)
                buf[r, pl.ds(0, L)] = buf[r, pl.ds(0, L)] * wv
                buf[r, pl.ds(L, L)] = buf[r, pl.ds(L, L)] * wv
            return 0
        lax.fori_loop(0, CHUNK // L, _m, 0)

    def add_rows(dst, src, nrows):
        def _a(r, _):
            dst[r, pl.ds(0, L)] = dst[r, pl.ds(0, L)] + src[r, pl.ds(0, L)]
            dst[r, pl.ds(L, L)] = dst[r, pl.ds(L, L)] + src[r, pl.ds(L, L)]
            return 0
        lax.fori_loop(0, nrows, _a, 0)

    tables = [emb_cat, ping_a, ping_b]
    for k in range(N_LAYERS):
        table = tables[k]

        def _zb(r, _):
            z = jnp.zeros((L,), f32)
            rows[r, pl.ds(0, L)] = z
            rows[r, pl.ds(L, L)] = z
            return 0
        lax.fori_loop(0, ZR, _zb, 0)

        zd = []
        for z in range(ROWS_PER_TILE // ZR):
            zd.append(pltpu.async_copy(
                rows.at[pl.ds(0, ZR)],
                acc.at[pl.ds(base_row + z * ZR, ZR)], ssem))
        pltpu.sync_copy(pack.at[c, pl.ds(cbase, IB)], idx_blk.at[0])
        pltpu.sync_copy(wp.at[pl.ds(cbase, IB)], w_blk.at[0])
        for d in zd:
            d.wait()
        plsc.subcore_barrier()

        def _blk(ob, _):
            p = ob % 2
            q = (ob + 1) % 2
            co = cbase + ob * IB

            @pl.when(ob + 1 < OUTER)
            def _pf():
                pltpu.async_copy(pack.at[c, pl.ds(co + IB, IB)],
                                 idx_blk.at[q], isem)
                pltpu.async_copy(wp.at[pl.ds(co + IB, IB)], w_blk.at[q], isem)

            gd = {}
            sd = {}
            gd[0] = pltpu.async_copy(table.at[idx_blk.at[p, 0, 0]], bufs[0], gsem)
            gd[1] = pltpu.async_copy(table.at[idx_blk.at[p, 1, 0]], bufs[1], gsem)
            for j in range(IB):
                gd[j].wait()
                if j + 2 < IB:
                    if j >= 2:
                        sd[j - 2].wait()
                    gd[j + 2] = pltpu.async_copy(
                        table.at[idx_blk.at[p, j + 2, 0]], bufs[(j + 2) % 4], gsem)
                scale(p, j, bufs[j % 4])
                sd[j] = pltpu.async_copy(
                    bufs[j % 4], acc.at[idx_blk.at[p, j, 1]], ssem, add=True)
            sd[IB - 4].wait()
            sd[IB - 3].wait()
            sd[IB - 2].wait()
            sd[IB - 1].wait()

            @pl.when(ob + 1 < OUTER)
            def _pfw():
                pltpu.make_async_copy(pack.at[c, pl.ds(co + IB, IB)],
                                      idx_blk.at[q], isem).wait()
                pltpu.make_async_copy(wp.at[pl.ds(co + IB, IB)],
                                      w_blk.at[q], isem).wait()
            return 0
        lax.fori_loop(0, OUTER, _blk, 0)
        plsc.subcore_barrier()

        out_tab = (ping_a, ping_b, ping_c)[k]
        pltpu.sync_copy(
            acc.at[pl.ds(base_row, ROWS_PER_TILE)],
            out_tab.at[pl.ds(c * NP + base_row, ROWS_PER_TILE)])
        plsc.subcore_barrier()

    lane_iota = lax.iota(jnp.int32, L)
    perms = [lane_iota ^ kk for kk in (8, 4, 2, 1)]

    def hsum(v):
        for p in perms:
            v = v + jnp.take(v, p)
        return v

    def acc_rows(dst, srcb, first):
        def _a(r, _):
            if first:
                dst[r, pl.ds(0, L)] = srcb[r, pl.ds(0, L)]
                dst[r, pl.ds(L, L)] = srcb[r, pl.ds(L, L)]
            else:
                dst[r, pl.ds(0, L)] = dst[r, pl.ds(0, L)] + srcb[r, pl.ds(0, L)]
                dst[r, pl.ds(L, L)] = dst[r, pl.ds(L, L)] + srcb[r, pl.ds(L, L)]
            return 0
        lax.fori_loop(0, CHUNK, _a, 0)

    for h in range(UH):
        for ti, tab in enumerate((emb_cat, ping_a, ping_b, ping_c)):
            du = pltpu.async_copy(tab.at[u_idx.at[h]], rows, gsem)
            di = pltpu.async_copy(tab.at[i_idx.at[h]], rows2, ssem)
            du.wait()
            di.wait()
            acc_rows(rows4, rows, ti == 0)
            acc_rows(rows3, rows2, ti == 0)

        def _g(g, _):
            acc16 = jnp.zeros((L,), f32)
            for lane in range(L):
                b2 = g * L + lane
                s = (rows4[b2, pl.ds(0, L)] * rows3[b2, pl.ds(0, L)]
                     + rows4[b2, pl.ds(L, L)] * rows3[b2, pl.ds(L, L)])
                sv = hsum(s) * f32(1.0 / 16.0)
                acc16 = jnp.where(lane_iota == lane, sv, acc16)
            gamma_v[pl.ds(h * CHUNK + g * L, L)] = acc16
            return 0
        lax.fori_loop(0, CHUNK // L, _g, 0)
    pltpu.sync_copy(gamma_v, gamma_out.at[c, pl.ds(t * BT, BT)])


@jax.jit
def _sc_call(emb_cat, pack, wp, uix, iix):
    mesh = plsc.VectorSubcoreMesh(core_axis_name="c", subcore_axis_name="s")
    f32 = jnp.float32
    call = pl.kernel(
        _body,
        out_type=[
            jax.ShapeDtypeStruct((NC, B), f32),
            jax.ShapeDtypeStruct((NC * NP, DH), f32),
            jax.ShapeDtypeStruct((NC * NP, DH), f32),
            jax.ShapeDtypeStruct((NC * NP, DH), f32),
        ],
        mesh=mesh,
        compiler_params=pltpu.CompilerParams(use_tc_tiling_on_sc=False),
        scratch_types=[
            pltpu.VMEM_SHARED((NP, DH), f32),
            pltpu.VMEM((2, IB, 2, CHUNK), jnp.int32),
            pltpu.VMEM((2, IB, CHUNK), f32),
            pltpu.VMEM((CHUNK, DH), f32),
            pltpu.VMEM((CHUNK, DH), f32),
            pltpu.VMEM((CHUNK, DH), f32),
            pltpu.VMEM((CHUNK, DH), f32),
            pltpu.VMEM((UH, CHUNK), jnp.int32),
            pltpu.VMEM((UH, CHUNK), jnp.int32),
            pltpu.VMEM((BT,), f32),
            pltpu.SemaphoreType.DMA,
            pltpu.SemaphoreType.DMA,
            pltpu.SemaphoreType.DMA,
        ],
    )
    return call(emb_cat, pack, wp, uix, iix)


def kernel(user_emb, item_emb, edge_weight, users, items, edge_index):
    all_emb = jnp.concatenate([user_emb, item_emb], axis=0)
    zrows = jnp.zeros((NP - N, DH), jnp.float32)
    emb_cat = jnp.concatenate(
        [all_emb[:, :DH], zrows, all_emb[:, DH:], zrows], axis=0)

    src = edge_index[0]
    dst = edge_index[1]
    pad = E_PAD - E
    srcp = jnp.pad(src, (0, pad))
    dstp = jnp.pad(dst, (0, pad)).reshape(CH, CHUNK)
    wp = jnp.pad(edge_weight, (0, pad)).reshape(CH, CHUNK)
    srcp = jnp.stack([srcp, srcp + NP]).reshape(NC, CH, CHUNK)
    pack = jnp.stack(
        [srcp, jnp.broadcast_to(dstp, (NC, CH, CHUNK))], axis=2)

    uix = jnp.stack([users, users + NP]).reshape(NC, B // CHUNK, CHUNK)
    it = items + N_USERS
    iix = jnp.stack([it, it + NP]).reshape(NC, B // CHUNK, CHUNK)

    gamma_parts, _, _, _ = _sc_call(emb_cat, pack, wp, uix, iix)
    return gamma_parts[0] + gamma_parts[1]


# speedup vs baseline: 2.3756x; 1.0169x over previous
import jax
import jax.numpy as jnp
from jax import lax
from jax.experimental import pallas as pl
from jax.experimental.pallas import tpu as pltpu
from jax.experimental.pallas import tpu_sc as plsc

N_USERS = 25000
N_ITEMS = 25000
N = N_USERS + N_ITEMS
D = 64
DH = D // 2
N_LAYERS = 3
E = 800000
B = 4096

NC = 2
NS = 16
L = 16

CHUNK = 128
OUTER = 49
IB = 8
CT = OUTER * IB
CH = CT * NS
E_PAD = CH * CHUNK
NP = 50176
ROWS_PER_TILE = NP // NS
ZR = 112
BT = B // NS
UH = BT // CHUNK


def _body(emb_cat, pack, wp, uix, iix,
          gamma_out, ping_a, ping_b, ping_c,
          acc,
          idx_blk, w_blk, rows, rows2, rows3, rows4,
          u_idx, i_idx, gamma_v, gsem, ssem, isem):
    c = lax.axis_index("c")
    t = lax.axis_index("s")
    f32 = jnp.float32
    bufs = (rows, rows2, rows3, rows4)

    pltpu.sync_copy(uix.at[c, pl.ds(UH * t, UH)], u_idx)
    pltpu.sync_copy(iix.at[c, pl.ds(UH * t, UH)], i_idx)

    base_row = t * ROWS_PER_TILE
    cbase = t * CT

    def scale(p, j, buf):
        def _m(g, _):
            w16 = w_blk[p, j, pl.ds(g * L, L)]
            for lane in range(L):
                r = g * L + lane
                wv = jnp.full((L,), w16[lane], dtype=f32)
                buf[r, pl.ds(0, L)] = buf[r, pl.ds(0, L)] * wv
                buf[r, pl.ds(L, L)] = buf[r, pl.ds(L, L)] * wv
            return 0
        lax.fori_loop(0, CHUNK // L, _m, 0)

    def add_rows(dst, src, nrows):
        def _a(r, _):
            dst[r, pl.ds(0, L)] = dst[r, pl.ds(0, L)] + src[r, pl.ds(0, L)]
            dst[r, pl.ds(L, L)] = dst[r, pl.ds(L, L)] + src[r, pl.ds(L, L)]
            return 0
        lax.fori_loop(0, nrows, _a, 0)

    tables = [emb_cat, ping_a, ping_b]
    for k in range(N_LAYERS):
        table = tables[k]

        def _zb(r, _):
            z = jnp.zeros((L,), f32)
            rows[r, pl.ds(0, L)] = z
            rows[r, pl.ds(L, L)] = z
            return 0
        lax.fori_loop(0, ZR, _zb, 0)

        zd = []
        for z in range(ROWS_PER_TILE // ZR):
            zd.append(pltpu.async_copy(
                rows.at[pl.ds(0, ZR)],
                acc.at[pl.ds(base_row + z * ZR, ZR)], ssem))
        pltpu.sync_copy(pack.at[c, pl.ds(cbase, IB)], idx_blk.at[0])
        pltpu.sync_copy(wp.at[pl.ds(cbase, IB)], w_blk.at[0])
        for d in zd:
            d.wait()
        plsc.subcore_barrier()

        def _blk(ob, _):
            p = ob % 2
            q = (ob + 1) % 2
            co = cbase + ob * IB

            @pl.when(ob + 1 < OUTER)
            def _pf():
                pltpu.async_copy(pack.at[c, pl.ds(co + IB, IB)],
                                 idx_blk.at[q], isem)
                pltpu.async_copy(wp.at[pl.ds(co + IB, IB)], w_blk.at[q], isem)

            gd = {}
            sd = {}
            gd[0] = pltpu.async_copy(table.at[idx_blk.at[p, 0, 0]], bufs[0], gsem)
            gd[1] = pltpu.async_copy(table.at[idx_blk.at[p, 1, 0]], bufs[1], gsem)
            gd[2] = pltpu.async_copy(table.at[idx_blk.at[p, 2, 0]], bufs[2], gsem)
            for j in range(IB):
                gd[j].wait()
                if j + 3 < IB:
                    if j >= 1:
                        sd[j - 1].wait()
                    gd[j + 3] = pltpu.async_copy(
                        table.at[idx_blk.at[p, j + 3, 0]], bufs[(j + 3) % 4], gsem)
                scale(p, j, bufs[j % 4])
                sd[j] = pltpu.async_copy(
                    bufs[j % 4], acc.at[idx_blk.at[p, j, 1]], ssem, add=True)
            sd[IB - 4].wait()
            sd[IB - 3].wait()
            sd[IB - 2].wait()
            sd[IB - 1].wait()

            @pl.when(ob + 1 < OUTER)
            def _pfw():
                pltpu.make_async_copy(pack.at[c, pl.ds(co + IB, IB)],
                                      idx_blk.at[q], isem).wait()
                pltpu.make_async_copy(wp.at[pl.ds(co + IB, IB)],
                                      w_blk.at[q], isem).wait()
            return 0
        lax.fori_loop(0, OUTER, _blk, 0)
        plsc.subcore_barrier()

        out_tab = (ping_a, ping_b, ping_c)[k]
        pltpu.sync_copy(
            acc.at[pl.ds(base_row, ROWS_PER_TILE)],
            out_tab.at[pl.ds(c * NP + base_row, ROWS_PER_TILE)])
        plsc.subcore_barrier()

    lane_iota = lax.iota(jnp.int32, L)
    perms = [lane_iota ^ kk for kk in (8, 4, 2, 1)]

    def hsum(v):
        for p in perms:
            v = v + jnp.take(v, p)
        return v

    def acc_rows(dst, srcb, first):
        def _a(r, _):
            if first:
                dst[r, pl.ds(0, L)] = srcb[r, pl.ds(0, L)]
                dst[r, pl.ds(L, L)] = srcb[r, pl.ds(L, L)]
            else:
                dst[r, pl.ds(0, L)] = dst[r, pl.ds(0, L)] + srcb[r, pl.ds(0, L)]
                dst[r, pl.ds(L, L)] = dst[r, pl.ds(L, L)] + srcb[r, pl.ds(L, L)]
            return 0
        lax.fori_loop(0, CHUNK, _a, 0)

    for h in range(UH):
        for ti, tab in enumerate((emb_cat, ping_a, ping_b, ping_c)):
            du = pltpu.async_copy(tab.at[u_idx.at[h]], rows, gsem)
            di = pltpu.async_copy(tab.at[i_idx.at[h]], rows2, ssem)
            du.wait()
            di.wait()
            acc_rows(rows4, rows, ti == 0)
            acc_rows(rows3, rows2, ti == 0)

        def _g(g, _):
            acc16 = jnp.zeros((L,), f32)
            for lane in range(L):
                b2 = g * L + lane
                s = (rows4[b2, pl.ds(0, L)] * rows3[b2, pl.ds(0, L)]
                     + rows4[b2, pl.ds(L, L)] * rows3[b2, pl.ds(L, L)])
                sv = hsum(s) * f32(1.0 / 16.0)
                acc16 = jnp.where(lane_iota == lane, sv, acc16)
            gamma_v[pl.ds(h * CHUNK + g * L, L)] = acc16
            return 0
        lax.fori_loop(0, CHUNK // L, _g, 0)
    pltpu.sync_copy(gamma_v, gamma_out.at[c, pl.ds(t * BT, BT)])


@jax.jit
def _sc_call(emb_cat, pack, wp, uix, iix):
    mesh = plsc.VectorSubcoreMesh(core_axis_name="c", subcore_axis_name="s")
    f32 = jnp.float32
    call = pl.kernel(
        _body,
        out_type=[
            jax.ShapeDtypeStruct((NC, B), f32),
            jax.ShapeDtypeStruct((NC * NP, DH), f32),
            jax.ShapeDtypeStruct((NC * NP, DH), f32),
            jax.ShapeDtypeStruct((NC * NP, DH), f32),
        ],
        mesh=mesh,
        compiler_params=pltpu.CompilerParams(use_tc_tiling_on_sc=False),
        scratch_types=[
            pltpu.VMEM_SHARED((NP, DH), f32),
            pltpu.VMEM((2, IB, 2, CHUNK), jnp.int32),
            pltpu.VMEM((2, IB, CHUNK), f32),
            pltpu.VMEM((CHUNK, DH), f32),
            pltpu.VMEM((CHUNK, DH), f32),
            pltpu.VMEM((CHUNK, DH), f32),
            pltpu.VMEM((CHUNK, DH), f32),
            pltpu.VMEM((UH, CHUNK), jnp.int32),
            pltpu.VMEM((UH, CHUNK), jnp.int32),
            pltpu.VMEM((BT,), f32),
            pltpu.SemaphoreType.DMA,
            pltpu.SemaphoreType.DMA,
            pltpu.SemaphoreType.DMA,
        ],
    )
    return call(emb_cat, pack, wp, uix, iix)


def kernel(user_emb, item_emb, edge_weight, users, items, edge_index):
    all_emb = jnp.concatenate([user_emb, item_emb], axis=0)
    zrows = jnp.zeros((NP - N, DH), jnp.float32)
    emb_cat = jnp.concatenate(
        [all_emb[:, :DH], zrows, all_emb[:, DH:], zrows], axis=0)

    src = edge_index[0]
    dst = edge_index[1]
    pad = E_PAD - E
    srcp = jnp.pad(src, (0, pad))
    dstp = jnp.pad(dst, (0, pad)).reshape(CH, CHUNK)
    wp = jnp.pad(edge_weight, (0, pad)).reshape(CH, CHUNK)
    srcp = jnp.stack([srcp, srcp + NP]).reshape(NC, CH, CHUNK)
    pack = jnp.stack(
        [srcp, jnp.broadcast_to(dstp, (NC, CH, CHUNK))], axis=2)

    uix = jnp.stack([users, users + NP]).reshape(NC, B // CHUNK, CHUNK)
    it = items + N_USERS
    iix = jnp.stack([it, it + NP]).reshape(NC, B // CHUNK, CHUNK)

    gamma_parts, _, _, _ = _sc_call(emb_cat, pack, wp, uix, iix)
    return gamma_parts[0] + gamma_parts[1]


# speedup vs baseline: 2.6522x; 1.1164x over previous
import jax
import jax.numpy as jnp
from jax import lax
from jax.experimental import pallas as pl
from jax.experimental.pallas import tpu as pltpu
from jax.experimental.pallas import tpu_sc as plsc

N_USERS = 25000
N_ITEMS = 25000
N = N_USERS + N_ITEMS
D = 64
DH = D // 2
N_LAYERS = 3
E = 800000
B = 4096

NC = 2
NS = 16
L = 16

CHUNK = 128
OUTER = 49
IB = 8
CT = OUTER * IB
CH = CT * NS
E_PAD = CH * CHUNK
NP = 50176
ROWS_PER_TILE = NP // NS
ZR = 112
BT = B // NS
UH = BT // CHUNK


def _body(emb_cat, pack, wp, uix, iix,
          gamma_out, ping_a, ping_b, ping_c,
          acc,
          idx_blk, w_blk, rows, rows2, rows3, rows4, rows5,
          u_idx, i_idx, gamma_v, gsem, ssem, isem):
    c = lax.axis_index("c")
    t = lax.axis_index("s")
    f32 = jnp.float32
    bufs = (rows, rows2, rows3, rows4, rows5)

    pltpu.sync_copy(uix.at[c, pl.ds(UH * t, UH)], u_idx)
    pltpu.sync_copy(iix.at[c, pl.ds(UH * t, UH)], i_idx)

    base_row = t * ROWS_PER_TILE
    cbase = t * CT

    def scale(p, j, buf):
        def _m(g, _):
            w16 = w_blk[p, j, pl.ds(g * L, L)]
            for lane in range(L):
                r = g * L + lane
                wv = jnp.full((L,), w16[lane], dtype=f32)
                buf[r, pl.ds(0, L)] = buf[r, pl.ds(0, L)] * wv
                buf[r, pl.ds(L, L)] = buf[r, pl.ds(L, L)] * wv
            return 0
        lax.fori_loop(0, CHUNK // L, _m, 0)

    def add_rows(dst, src, nrows):
        def _a(r, _):
            dst[r, pl.ds(0, L)] = dst[r, pl.ds(0, L)] + src[r, pl.ds(0, L)]
            dst[r, pl.ds(L, L)] = dst[r, pl.ds(L, L)] + src[r, pl.ds(L, L)]
            return 0
        lax.fori_loop(0, nrows, _a, 0)

    tables = [emb_cat, ping_a, ping_b]
    for k in range(N_LAYERS):
        table = tables[k]

        def _zb(r, _):
            z = jnp.zeros((L,), f32)
            rows[r, pl.ds(0, L)] = z
            rows[r, pl.ds(L, L)] = z
            return 0
        lax.fori_loop(0, ZR, _zb, 0)

        zd = []
        for z in range(ROWS_PER_TILE // ZR):
            zd.append(pltpu.async_copy(
                rows.at[pl.ds(0, ZR)],
                acc.at[pl.ds(base_row + z * ZR, ZR)], ssem))
        pltpu.sync_copy(pack.at[c, pl.ds(cbase, IB)], idx_blk.at[0])
        pltpu.sync_copy(wp.at[pl.ds(cbase, IB)], w_blk.at[0])
        for d in zd:
            d.wait()
        plsc.subcore_barrier()

        def _blk(ob, _):
            p = ob % 2
            q = (ob + 1) % 2
            co = cbase + ob * IB

            @pl.when(ob + 1 < OUTER)
            def _pf():
                pltpu.async_copy(pack.at[c, pl.ds(co + IB, IB)],
                                 idx_blk.at[q], isem)
                pltpu.async_copy(wp.at[pl.ds(co + IB, IB)], w_blk.at[q], isem)

            gd = {}
            sd = {}
            gd[0] = pltpu.async_copy(table.at[idx_blk.at[p, 0, 0]], bufs[0], gsem)
            gd[1] = pltpu.async_copy(table.at[idx_blk.at[p, 1, 0]], bufs[1], gsem)
            gd[2] = pltpu.async_copy(table.at[idx_blk.at[p, 2, 0]], bufs[2], gsem)
            for j in range(IB):
                gd[j].wait()
                if j + 3 < IB:
                    if j >= 2:
                        sd[j - 2].wait()
                    gd[j + 3] = pltpu.async_copy(
                        table.at[idx_blk.at[p, j + 3, 0]], bufs[(j + 3) % 5], gsem)
                scale(p, j, bufs[j % 5])
                sd[j] = pltpu.async_copy(
                    bufs[j % 5], acc.at[idx_blk.at[p, j, 1]], ssem, add=True)
            sd[IB - 4].wait()
            sd[IB - 3].wait()
            sd[IB - 2].wait()
            sd[IB - 1].wait()

            @pl.when(ob + 1 < OUTER)
            def _pfw():
                pltpu.make_async_copy(pack.at[c, pl.ds(co + IB, IB)],
                                      idx_blk.at[q], isem).wait()
                pltpu.make_async_copy(wp.at[pl.ds(co + IB, IB)],
                                      w_blk.at[q], isem).wait()
            return 0
        lax.fori_loop(0, OUTER, _blk, 0)
        plsc.subcore_barrier()

        out_tab = (ping_a, ping_b, ping_c)[k]
        pltpu.sync_copy(
            acc.at[pl.ds(base_row, ROWS_PER_TILE)],
            out_tab.at[pl.ds(c * NP + base_row, ROWS_PER_TILE)])
        plsc.subcore_barrier()

    lane_iota = lax.iota(jnp.int32, L)
    perms = [lane_iota ^ kk for kk in (8, 4, 2, 1)]

    def hsum(v):
        for p in perms:
            v = v + jnp.take(v, p)
        return v

    def acc_rows(dst, srcb, first):
        def _a(r, _):
            if first:
                dst[r, pl.ds(0, L)] = srcb[r, pl.ds(0, L)]
                dst[r, pl.ds(L, L)] = srcb[r, pl.ds(L, L)]
            else:
                dst[r, pl.ds(0, L)] = dst[r, pl.ds(0, L)] + srcb[r, pl.ds(0, L)]
                dst[r, pl.ds(L, L)] = dst[r, pl.ds(L, L)] + srcb[r, pl.ds(L, L)]
            return 0
        lax.fori_loop(0, CHUNK, _a, 0)

    for h in range(UH):
        for ti, tab in enumerate((emb_cat, ping_a, ping_b, ping_c)):
            du = pltpu.async_copy(tab.at[u_idx.at[h]], rows, gsem)
            di = pltpu.async_copy(tab.at[i_idx.at[h]], rows2, ssem)
            du.wait()
            di.wait()
            acc_rows(rows4, rows, ti == 0)
            acc_rows(rows3, rows2, ti == 0)

        def _g(g, _):
            acc16 = jnp.zeros((L,), f32)
            for lane in range(L):
                b2 = g * L + lane
                s = (rows4[b2, pl.ds(0, L)] * rows3[b2, pl.ds(0, L)]
                     + rows4[b2, pl.ds(L, L)] * rows3[b2, pl.ds(L, L)])
                sv = hsum(s) * f32(1.0 / 16.0)
                acc16 = jnp.where(lane_iota == lane, sv, acc16)
            gamma_v[pl.ds(h * CHUNK + g * L, L)] = acc16
            return 0
        lax.fori_loop(0, CHUNK // L, _g, 0)
    pltpu.sync_copy(gamma_v, gamma_out.at[c, pl.ds(t * BT, BT)])


@jax.jit
def _sc_call(emb_cat, pack, wp, uix, iix):
    mesh = plsc.VectorSubcoreMesh(core_axis_name="c", subcore_axis_name="s")
    f32 = jnp.float32
    call = pl.kernel(
        _body,
        out_type=[
            jax.ShapeDtypeStruct((NC, B), f32),
            jax.ShapeDtypeStruct((NC * NP, DH), f32),
            jax.ShapeDtypeStruct((NC * NP, DH), f32),
            jax.ShapeDtypeStruct((NC * NP, DH), f32),
        ],
        mesh=mesh,
        compiler_params=pltpu.CompilerParams(use_tc_tiling_on_sc=False),
        scratch_types=[
            pltpu.VMEM_SHARED((NP, DH), f32),
            pltpu.VMEM((2, IB, 2, CHUNK), jnp.int32),
            pltpu.VMEM((2, IB, CHUNK), f32),
            pltpu.VMEM((CHUNK, DH), f32),
            pltpu.VMEM((CHUNK, DH), f32),
            pltpu.VMEM((CHUNK, DH), f32),
            pltpu.VMEM((CHUNK, DH), f32),
            pltpu.VMEM((CHUNK, DH), f32),
            pltpu.VMEM((UH, CHUNK), jnp.int32),
            pltpu.VMEM((UH, CHUNK), jnp.int32),
            pltpu.VMEM((BT,), f32),
            pltpu.SemaphoreType.DMA,
            pltpu.SemaphoreType.DMA,
            pltpu.SemaphoreType.DMA,
        ],
    )
    return call(emb_cat, pack, wp, uix, iix)


def kernel(user_emb, item_emb, edge_weight, users, items, edge_index):
    all_emb = jnp.concatenate([user_emb, item_emb], axis=0)
    zrows = jnp.zeros((NP - N, DH), jnp.float32)
    emb_cat = jnp.concatenate(
        [all_emb[:, :DH], zrows, all_emb[:, DH:], zrows], axis=0)

    src = edge_index[0]
    dst = edge_index[1]
    pad = E_PAD - E
    srcp = jnp.pad(src, (0, pad))
    dstp = jnp.pad(dst, (0, pad)).reshape(CH, CHUNK)
    wp = jnp.pad(edge_weight, (0, pad)).reshape(CH, CHUNK)
    srcp = jnp.stack([srcp, srcp + NP]).reshape(NC, CH, CHUNK)
    pack = jnp.stack(
        [srcp, jnp.broadcast_to(dstp, (NC, CH, CHUNK))], axis=2)

    uix = jnp.stack([users, users + NP]).reshape(NC, B // CHUNK, CHUNK)
    it = items + N_USERS
    iix = jnp.stack([it, it + NP]).reshape(NC, B // CHUNK, CHUNK)

    gamma_parts, _, _, _ = _sc_call(emb_cat, pack, wp, uix, iix)
    return gamma_parts[0] + gamma_parts[1]
